# baseline (device time: 2434 ns/iter reference)
import jax
import jax.numpy as jnp
from jax import lax
from jax.experimental import pallas as pl
from jax.experimental.pallas import tpu as pltpu

M_GLOBAL = 2048


def kernel(x):
    m_per, n_per = x.shape

    def body(x_ref, out_ref):
        ones = jnp.ones((8, m_per), jnp.float32)
        s = jnp.dot(ones, x_ref[:, :], preferred_element_type=jnp.float32)
        out_ref[:, :] = s[0:1, :] * (1.0 / M_GLOBAL)

    return pl.pallas_call(
        body,
        out_shape=jax.ShapeDtypeStruct((1, n_per), jnp.float32),
        in_specs=[pl.BlockSpec(memory_space=pltpu.VMEM)],
        out_specs=pl.BlockSpec(memory_space=pltpu.VMEM),
    )(x)
